# Initial kernel scaffold; baseline (speedup 1.0000x reference)
import jax
import jax.numpy as jnp
from jax.experimental import pallas as pl


def kernel(x, edge_index, y, W1, b1, W2, b2, Wc1, bc1, Wc2, bc2, gamma, beta):
    raise NotImplementedError("write your pallas kernel here")


# speedup vs baseline: 20.2617x; 20.2617x over previous
import functools

import jax
import jax.numpy as jnp
from jax import lax
from jax.experimental import pallas as pl
from jax.experimental.pallas import tpu as pltpu
from jax.experimental.pallas import tpu_sc as plsc

N = 10000
D = 128
H = 64
C = 40

NC = 2
NS = 16
NW = NC * NS

NPAD = 10240
ROWS_PER_TILE = NPAD // NS
CHUNK = 128

BN = 512
GRID = NPAD // BN


def _mesh():
    return plsc.VectorSubcoreMesh(core_axis_name="c", subcore_axis_name="s")


_SC_PARAMS = pltpu.CompilerParams(use_tc_tiling_on_sc=False)


def _make_deg_kernel(n_chunks):
    @functools.partial(
        pl.kernel,
        out_type=jax.ShapeDtypeStruct((NC, NPAD), jnp.float32),
        mesh=_mesh(),
        compiler_params=_SC_PARAMS,
        scratch_types=dict(
            dst_v=pltpu.VMEM((n_chunks, CHUNK), jnp.int32),
            ones_v=pltpu.VMEM((CHUNK,), jnp.float32),
            zbuf=pltpu.VMEM((ROWS_PER_TILE,), jnp.float32),
            acc=pltpu.VMEM_SHARED((NPAD,), jnp.float32),
            sem=pltpu.SemaphoreType.DMA,
        ),
    )
    def deg_kernel(dst_hbm, deg_out, dst_v, ones_v, zbuf, acc, sem):
        cid = lax.axis_index("c")
        sid = lax.axis_index("s")
        wid = sid * NC + cid

        def fill_ones(i, _):
            ones_v[pl.ds(i * 16, 16)] = jnp.ones((16,), jnp.float32)
            zbuf[pl.ds(i * 16, 16)] = jnp.zeros((16,), jnp.float32)
            return 0

        lax.fori_loop(0, CHUNK // 16, fill_ones, 0)

        def fill_z(i, _):
            zbuf[pl.ds(i * 16, 16)] = jnp.zeros((16,), jnp.float32)
            return 0

        lax.fori_loop(CHUNK // 16, ROWS_PER_TILE // 16, fill_z, 0)

        copy_idx = pltpu.async_copy(dst_hbm.at[wid], dst_v, sem)
        pltpu.sync_copy(zbuf, acc.at[pl.ds(sid * ROWS_PER_TILE, ROWS_PER_TILE)])
        copy_idx.wait()
        plsc.subcore_barrier()

        def body(j, _):
            pltpu.sync_copy(ones_v, acc.at[dst_v.at[j]], add=True)
            return 0

        lax.fori_loop(0, n_chunks, body, 0)
        plsc.subcore_barrier()

        pltpu.sync_copy(
            acc.at[pl.ds(sid * ROWS_PER_TILE, ROWS_PER_TILE)],
            deg_out.at[cid, pl.ds(sid * ROWS_PER_TILE, ROWS_PER_TILE)],
        )

    return deg_kernel


def _make_agg_kernel(n_chunks):
    @functools.partial(
        pl.kernel,
        out_type=jax.ShapeDtypeStruct((NC, NPAD, H), jnp.float32),
        mesh=_mesh(),
        compiler_params=_SC_PARAMS,
        scratch_types=dict(
            src_v=pltpu.VMEM((n_chunks, CHUNK), jnp.int32),
            dst_v=pltpu.VMEM((n_chunks, CHUNK), jnp.int32),
            rows_v=pltpu.VMEM((2, CHUNK, H), jnp.float32),
            zbuf=pltpu.VMEM((ROWS_PER_TILE, H), jnp.float32),
            acc=pltpu.VMEM_SHARED((NPAD, H), jnp.float32),
            sem_g=pltpu.SemaphoreType.DMA,
            sem_i=pltpu.SemaphoreType.DMA,
        ),
    )
    def agg_kernel(hs_hbm, src_hbm, dst_hbm, p_out,
                   src_v, dst_v, rows_v, zbuf, acc, sem_g, sem_i):
        cid = lax.axis_index("c")
        sid = lax.axis_index("s")
        wid = sid * NC + cid

        c_src = pltpu.async_copy(src_hbm.at[wid], src_v, sem_i)
        c_dst = pltpu.async_copy(dst_hbm.at[wid], dst_v, sem_i)

        def fill_z(r, _):
            for c4 in range(H // 16):
                zbuf[r, pl.ds(c4 * 16, 16)] = jnp.zeros((16,), jnp.float32)
            return 0

        lax.fori_loop(0, ROWS_PER_TILE, fill_z, 0)
        pltpu.sync_copy(zbuf, acc.at[pl.ds(sid * ROWS_PER_TILE,
                                           ROWS_PER_TILE)])
        c_src.wait()
        c_dst.wait()
        plsc.subcore_barrier()

        pltpu.async_copy(hs_hbm.at[src_v.at[0]], rows_v.at[0], sem_g).wait()

        def body(j, _):
            slot = lax.rem(j, 2)
            nxt = lax.rem(j + 1, 2)
            nxt_gather = pltpu.async_copy(
                hs_hbm.at[src_v.at[lax.min(j + 1, n_chunks - 1)]],
                rows_v.at[nxt], sem_g)
            pltpu.sync_copy(rows_v.at[slot], acc.at[dst_v.at[j]], add=True)
            nxt_gather.wait()
            return 0

        lax.fori_loop(0, n_chunks, body, 0)
        plsc.subcore_barrier()

        pltpu.sync_copy(
            acc.at[pl.ds(sid * ROWS_PER_TILE, ROWS_PER_TILE)],
            p_out.at[cid, pl.ds(sid * ROWS_PER_TILE, ROWS_PER_TILE)],
        )

    return agg_kernel


def _tc1_body(deg_ref, x_ref, w_ref, dinv_ref, hs_ref):
    d = deg_ref[0] + deg_ref[1] + 1.0
    dinv = lax.rsqrt(d)
    dinv_ref[...] = dinv
    h = jnp.dot(x_ref[...], w_ref[...], preferred_element_type=jnp.float32)
    hs_ref[...] = h * dinv


def _tc2_body(p_ref, hs_ref, dinv_ref, b1_ref, w2_ref, out_ref):
    dinv = dinv_ref[...]
    agg = (p_ref[0] + p_ref[1] + hs_ref[...]) * dinv + b1_ref[...]
    h1 = jnp.maximum(agg, 0.0)
    out_ref[...] = jnp.dot(h1, w2_ref[...],
                           preferred_element_type=jnp.float32) * dinv


def _tc3_body(p_ref, hs_ref, dinv_ref, b2_ref, wc1_ref, bc1_ref, wc2_ref,
              bc2_ref, g_ref, bt_ref, y_ref, loss_ref, acc_ref):
    i = pl.program_id(0)
    h2 = (p_ref[0] + p_ref[1] + hs_ref[...]) * dinv_ref[...] + b2_ref[...]
    t = jnp.maximum(jnp.dot(h2, wc1_ref[...],
                            preferred_element_type=jnp.float32)
                    + bc1_ref[...], 0.0)
    o = jnp.dot(t, wc2_ref[...],
                preferred_element_type=jnp.float32) + bc2_ref[...]
    mu = jnp.mean(o, axis=1, keepdims=True)
    ctr = o - mu
    var = jnp.mean(ctr * ctr, axis=1, keepdims=True)
    o = ctr * lax.rsqrt(var + 1e-5) * g_ref[...] + bt_ref[...]
    m = jnp.max(o, axis=1, keepdims=True)
    lse = m + jnp.log(jnp.sum(jnp.exp(o - m), axis=1, keepdims=True))
    cls = lax.broadcasted_iota(jnp.int32, (BN, C), 1)
    tgt = jnp.sum(jnp.where(cls == y_ref[...], o, 0.0), axis=1, keepdims=True)
    rows = i * BN + lax.broadcasted_iota(jnp.int32, (BN, 1), 0)
    part = jnp.sum(jnp.where(rows < N, tgt - lse, 0.0))
    tot = jnp.where(i == 0, 0.0, acc_ref[0, 0]) + part
    acc_ref[0, 0] = tot

    @pl.when(i == GRID - 1)
    def _():
        loss_ref[0, 0] = -tot / N


def kernel(x, edge_index, y, W1, b1, W2, b2, Wc1, bc1, Wc2, bc2, gamma, beta):
    e = edge_index.shape[1]
    e_per_tile = -(-e // (NW * CHUNK)) * CHUNK
    n_chunks = e_per_tile // CHUNK
    epad = e_per_tile * NW

    x_pad = jnp.zeros((NPAD, D), jnp.float32).at[:N].set(x)
    src = jnp.full((epad,), NPAD - 1, jnp.int32).at[:e].set(
        edge_index[0].astype(jnp.int32))
    dst = jnp.full((epad,), NPAD - 1, jnp.int32).at[:e].set(
        edge_index[1].astype(jnp.int32))
    src_r = src.reshape(NW, n_chunks, CHUNK)
    dst_r = dst.reshape(NW, n_chunks, CHUNK)
    y_pad = jnp.zeros((NPAD, 1), jnp.int32).at[:N].set(y.astype(jnp.int32))

    deg_p = _make_deg_kernel(n_chunks)(dst_r)
    deg_p = deg_p.reshape(NC, NPAD, 1)

    dinv, hs1 = pl.pallas_call(
        _tc1_body,
        grid=(GRID,),
        in_specs=[
            pl.BlockSpec((NC, BN, 1), lambda i: (0, i, 0)),
            pl.BlockSpec((BN, D), lambda i: (i, 0)),
            pl.BlockSpec((D, H), lambda i: (0, 0)),
        ],
        out_specs=[
            pl.BlockSpec((BN, 1), lambda i: (i, 0)),
            pl.BlockSpec((BN, H), lambda i: (i, 0)),
        ],
        out_shape=[
            jax.ShapeDtypeStruct((NPAD, 1), jnp.float32),
            jax.ShapeDtypeStruct((NPAD, H), jnp.float32),
        ],
    )(deg_p, x_pad, W1)

    agg = _make_agg_kernel(n_chunks)

    p1 = agg(hs1, src_r, dst_r)

    hs2 = pl.pallas_call(
        _tc2_body,
        grid=(GRID,),
        in_specs=[
            pl.BlockSpec((NC, BN, H), lambda i: (0, i, 0)),
            pl.BlockSpec((BN, H), lambda i: (i, 0)),
            pl.BlockSpec((BN, 1), lambda i: (i, 0)),
            pl.BlockSpec((1, H), lambda i: (0, 0)),
            pl.BlockSpec((H, H), lambda i: (0, 0)),
        ],
        out_specs=pl.BlockSpec((BN, H), lambda i: (i, 0)),
        out_shape=jax.ShapeDtypeStruct((NPAD, H), jnp.float32),
    )(p1, hs1, dinv, b1.reshape(1, H), W2)

    p2 = agg(hs2, src_r, dst_r)

    loss = pl.pallas_call(
        _tc3_body,
        grid=(GRID,),
        in_specs=[
            pl.BlockSpec((NC, BN, H), lambda i: (0, i, 0)),
            pl.BlockSpec((BN, H), lambda i: (i, 0)),
            pl.BlockSpec((BN, 1), lambda i: (i, 0)),
            pl.BlockSpec((1, H), lambda i: (0, 0)),
            pl.BlockSpec((H, H), lambda i: (0, 0)),
            pl.BlockSpec((1, H), lambda i: (0, 0)),
            pl.BlockSpec((H, C), lambda i: (0, 0)),
            pl.BlockSpec((1, C), lambda i: (0, 0)),
            pl.BlockSpec((1, C), lambda i: (0, 0)),
            pl.BlockSpec((1, C), lambda i: (0, 0)),
            pl.BlockSpec((BN, 1), lambda i: (i, 0)),
        ],
        out_specs=pl.BlockSpec((1, 1), lambda i: (0, 0),
                               memory_space=pltpu.SMEM),
        out_shape=jax.ShapeDtypeStruct((1, 1), jnp.float32),
        scratch_shapes=[pltpu.SMEM((1, 1), jnp.float32)],
    )(p2, hs2, dinv, b2.reshape(1, H), Wc1, bc1.reshape(1, H), Wc2,
      bc2.reshape(1, C), gamma.reshape(1, C), beta.reshape(1, C), y_pad)

    return loss[0, 0]


# speedup vs baseline: 23.7648x; 1.1729x over previous
import functools

import jax
import jax.numpy as jnp
from jax import lax
from jax.experimental import pallas as pl
from jax.experimental.pallas import tpu as pltpu
from jax.experimental.pallas import tpu_sc as plsc

N = 10000
D = 128
H = 64
C = 40

NC = 2
NS = 16
NW = NC * NS

NPAD = 10240
ROWS_PER_TILE = NPAD // NS
CHUNK = 128

BN = 512
GRID = NPAD // BN


def _mesh():
    return plsc.VectorSubcoreMesh(core_axis_name="c", subcore_axis_name="s")


_SC_PARAMS = pltpu.CompilerParams(use_tc_tiling_on_sc=False)


def _make_deg_kernel(n_chunks):
    @functools.partial(
        pl.kernel,
        out_type=jax.ShapeDtypeStruct((NC, NPAD), jnp.float32),
        mesh=_mesh(),
        compiler_params=_SC_PARAMS,
        scratch_types=dict(
            dst_v=pltpu.VMEM((n_chunks, CHUNK), jnp.int32),
            ones_v=pltpu.VMEM((CHUNK,), jnp.float32),
            zbuf=pltpu.VMEM((ROWS_PER_TILE,), jnp.float32),
            acc=pltpu.VMEM_SHARED((NPAD,), jnp.float32),
            sem=pltpu.SemaphoreType.DMA,
        ),
    )
    def deg_kernel(dst_hbm, deg_out, dst_v, ones_v, zbuf, acc, sem):
        cid = lax.axis_index("c")
        sid = lax.axis_index("s")
        wid = sid * NC + cid

        def fill_ones(i, _):
            ones_v[pl.ds(i * 16, 16)] = jnp.ones((16,), jnp.float32)
            zbuf[pl.ds(i * 16, 16)] = jnp.zeros((16,), jnp.float32)
            return 0

        lax.fori_loop(0, CHUNK // 16, fill_ones, 0)

        def fill_z(i, _):
            zbuf[pl.ds(i * 16, 16)] = jnp.zeros((16,), jnp.float32)
            return 0

        lax.fori_loop(CHUNK // 16, ROWS_PER_TILE // 16, fill_z, 0)

        copy_idx = pltpu.async_copy(dst_hbm.at[wid], dst_v, sem)
        pltpu.sync_copy(zbuf, acc.at[pl.ds(sid * ROWS_PER_TILE, ROWS_PER_TILE)])
        copy_idx.wait()
        plsc.subcore_barrier()

        def body(j, _):
            pltpu.sync_copy(ones_v, acc.at[dst_v.at[j]], add=True)
            return 0

        lax.fori_loop(0, n_chunks, body, 0)
        plsc.subcore_barrier()

        pltpu.sync_copy(
            acc.at[pl.ds(sid * ROWS_PER_TILE, ROWS_PER_TILE)],
            deg_out.at[cid, pl.ds(sid * ROWS_PER_TILE, ROWS_PER_TILE)],
        )

    return deg_kernel


def _make_agg_kernel(n_chunks):
    @functools.partial(
        pl.kernel,
        out_type=jax.ShapeDtypeStruct((NC, NPAD, H), jnp.float32),
        mesh=_mesh(),
        compiler_params=_SC_PARAMS,
        scratch_types=dict(
            src_v=pltpu.VMEM((n_chunks, CHUNK), jnp.int32),
            dst_v=pltpu.VMEM((n_chunks, CHUNK), jnp.int32),
            rows_v=pltpu.VMEM((4, CHUNK, H), jnp.float32),
            zbuf=pltpu.VMEM((CHUNK, H), jnp.float32),
            acc=pltpu.VMEM_SHARED((NPAD, H), jnp.float32),
            sem_g=pltpu.SemaphoreType.DMA,
            sem_s=pltpu.SemaphoreType.DMA,
            sem_i=pltpu.SemaphoreType.DMA,
        ),
    )
    def agg_kernel(hs_hbm, src_hbm, dst_hbm, p_out,
                   src_v, dst_v, rows_v, zbuf, acc, sem_g, sem_s, sem_i):
        cid = lax.axis_index("c")
        sid = lax.axis_index("s")
        wid = sid * NC + cid

        c_src = pltpu.async_copy(src_hbm.at[wid], src_v, sem_i)
        c_dst = pltpu.async_copy(dst_hbm.at[wid], dst_v, sem_i)

        def fill_z(r, _):
            for c4 in range(H // 16):
                zbuf[r, pl.ds(c4 * 16, 16)] = jnp.zeros((16,), jnp.float32)
            return 0

        lax.fori_loop(0, CHUNK, fill_z, 0)
        for k in range(ROWS_PER_TILE // CHUNK):
            pltpu.sync_copy(
                zbuf, acc.at[pl.ds(sid * ROWS_PER_TILE + k * CHUNK, CHUNK)])
        c_src.wait()
        c_dst.wait()
        plsc.subcore_barrier()

        def gather(j):
            pltpu.async_copy(hs_hbm.at[src_v.at[j]],
                             rows_v.at[lax.rem(j, 4)], sem_g)

        def scatter(j):
            pltpu.async_copy(rows_v.at[lax.rem(j, 4)],
                             acc.at[dst_v.at[j]], sem_s, add=True)

        def drain(sem):
            pltpu.make_async_copy(hs_hbm.at[src_v.at[0]], rows_v.at[0],
                                  sem).wait()

        for j0 in range(3):
            gather(j0)

        def body(j, _):
            drain(sem_g)
            scatter(j)

            @pl.when(j >= 1)
            def _():
                drain(sem_s)

            @pl.when(j + 3 < n_chunks)
            def _():
                gather(j + 3)
            return 0

        lax.fori_loop(0, n_chunks, body, 0)
        drain(sem_s)
        plsc.subcore_barrier()

        pltpu.sync_copy(
            acc.at[pl.ds(sid * ROWS_PER_TILE, ROWS_PER_TILE)],
            p_out.at[cid, pl.ds(sid * ROWS_PER_TILE, ROWS_PER_TILE)],
        )

    return agg_kernel


def _tc1_body(deg_ref, x_ref, w_ref, dinv_ref, hs_ref):
    d = deg_ref[0] + deg_ref[1] + 1.0
    dinv = lax.rsqrt(d)
    dinv_ref[...] = dinv
    h = jnp.dot(x_ref[...], w_ref[...], preferred_element_type=jnp.float32)
    hs_ref[...] = h * dinv


def _tc2_body(p_ref, hs_ref, dinv_ref, b1_ref, w2_ref, out_ref):
    dinv = dinv_ref[...]
    agg = (p_ref[0] + p_ref[1] + hs_ref[...]) * dinv + b1_ref[...]
    h1 = jnp.maximum(agg, 0.0)
    out_ref[...] = jnp.dot(h1, w2_ref[...],
                           preferred_element_type=jnp.float32) * dinv


def _tc3_body(p_ref, hs_ref, dinv_ref, b2_ref, wc1_ref, bc1_ref, wc2_ref,
              bc2_ref, g_ref, bt_ref, y_ref, loss_ref, acc_ref):
    i = pl.program_id(0)
    h2 = (p_ref[0] + p_ref[1] + hs_ref[...]) * dinv_ref[...] + b2_ref[...]
    t = jnp.maximum(jnp.dot(h2, wc1_ref[...],
                            preferred_element_type=jnp.float32)
                    + bc1_ref[...], 0.0)
    o = jnp.dot(t, wc2_ref[...],
                preferred_element_type=jnp.float32) + bc2_ref[...]
    mu = jnp.mean(o, axis=1, keepdims=True)
    ctr = o - mu
    var = jnp.mean(ctr * ctr, axis=1, keepdims=True)
    o = ctr * lax.rsqrt(var + 1e-5) * g_ref[...] + bt_ref[...]
    m = jnp.max(o, axis=1, keepdims=True)
    lse = m + jnp.log(jnp.sum(jnp.exp(o - m), axis=1, keepdims=True))
    cls = lax.broadcasted_iota(jnp.int32, (BN, C), 1)
    tgt = jnp.sum(jnp.where(cls == y_ref[...], o, 0.0), axis=1, keepdims=True)
    rows = i * BN + lax.broadcasted_iota(jnp.int32, (BN, 1), 0)
    part = jnp.sum(jnp.where(rows < N, tgt - lse, 0.0))
    tot = jnp.where(i == 0, 0.0, acc_ref[0, 0]) + part
    acc_ref[0, 0] = tot

    @pl.when(i == GRID - 1)
    def _():
        loss_ref[0, 0] = -tot / N


def kernel(x, edge_index, y, W1, b1, W2, b2, Wc1, bc1, Wc2, bc2, gamma, beta):
    e = edge_index.shape[1]
    e_per_tile = -(-e // (NW * CHUNK)) * CHUNK
    n_chunks = e_per_tile // CHUNK
    epad = e_per_tile * NW

    x_pad = jnp.zeros((NPAD, D), jnp.float32).at[:N].set(x)
    src = jnp.full((epad,), NPAD - 1, jnp.int32).at[:e].set(
        edge_index[0].astype(jnp.int32))
    dst = jnp.full((epad,), NPAD - 1, jnp.int32).at[:e].set(
        edge_index[1].astype(jnp.int32))
    src_r = src.reshape(NW, n_chunks, CHUNK)
    dst_r = dst.reshape(NW, n_chunks, CHUNK)
    y_pad = jnp.zeros((NPAD, 1), jnp.int32).at[:N].set(y.astype(jnp.int32))

    deg_p = _make_deg_kernel(n_chunks)(dst_r)
    deg_p = deg_p.reshape(NC, NPAD, 1)

    dinv, hs1 = pl.pallas_call(
        _tc1_body,
        grid=(GRID,),
        in_specs=[
            pl.BlockSpec((NC, BN, 1), lambda i: (0, i, 0)),
            pl.BlockSpec((BN, D), lambda i: (i, 0)),
            pl.BlockSpec((D, H), lambda i: (0, 0)),
        ],
        out_specs=[
            pl.BlockSpec((BN, 1), lambda i: (i, 0)),
            pl.BlockSpec((BN, H), lambda i: (i, 0)),
        ],
        out_shape=[
            jax.ShapeDtypeStruct((NPAD, 1), jnp.float32),
            jax.ShapeDtypeStruct((NPAD, H), jnp.float32),
        ],
    )(deg_p, x_pad, W1)

    agg = _make_agg_kernel(n_chunks)

    p1 = agg(hs1, src_r, dst_r)

    hs2 = pl.pallas_call(
        _tc2_body,
        grid=(GRID,),
        in_specs=[
            pl.BlockSpec((NC, BN, H), lambda i: (0, i, 0)),
            pl.BlockSpec((BN, H), lambda i: (i, 0)),
            pl.BlockSpec((BN, 1), lambda i: (i, 0)),
            pl.BlockSpec((1, H), lambda i: (0, 0)),
            pl.BlockSpec((H, H), lambda i: (0, 0)),
        ],
        out_specs=pl.BlockSpec((BN, H), lambda i: (i, 0)),
        out_shape=jax.ShapeDtypeStruct((NPAD, H), jnp.float32),
    )(p1, hs1, dinv, b1.reshape(1, H), W2)

    p2 = agg(hs2, src_r, dst_r)

    loss = pl.pallas_call(
        _tc3_body,
        grid=(GRID,),
        in_specs=[
            pl.BlockSpec((NC, BN, H), lambda i: (0, i, 0)),
            pl.BlockSpec((BN, H), lambda i: (i, 0)),
            pl.BlockSpec((BN, 1), lambda i: (i, 0)),
            pl.BlockSpec((1, H), lambda i: (0, 0)),
            pl.BlockSpec((H, H), lambda i: (0, 0)),
            pl.BlockSpec((1, H), lambda i: (0, 0)),
            pl.BlockSpec((H, C), lambda i: (0, 0)),
            pl.BlockSpec((1, C), lambda i: (0, 0)),
            pl.BlockSpec((1, C), lambda i: (0, 0)),
            pl.BlockSpec((1, C), lambda i: (0, 0)),
            pl.BlockSpec((BN, 1), lambda i: (i, 0)),
        ],
        out_specs=pl.BlockSpec((1, 1), lambda i: (0, 0),
                               memory_space=pltpu.SMEM),
        out_shape=jax.ShapeDtypeStruct((1, 1), jnp.float32),
        scratch_shapes=[pltpu.SMEM((1, 1), jnp.float32)],
    )(p2, hs2, dinv, b2.reshape(1, H), Wc1, bc1.reshape(1, H), Wc2,
      bc2.reshape(1, C), gamma.reshape(1, C), beta.reshape(1, C), y_pad)

    return loss[0, 0]


# speedup vs baseline: 36.2940x; 1.5272x over previous
import functools

import jax
import jax.numpy as jnp
from jax import lax
from jax.experimental import pallas as pl
from jax.experimental.pallas import tpu as pltpu
from jax.experimental.pallas import tpu_sc as plsc

N = 10000
D = 128
H = 64
C = 40

NC = 2
NS = 16
NW = NC * NS

NPAD = 10240
ROWS_PER_TILE = NPAD // NS
CHUNK = 128

BN = 512
GRID = NPAD // BN


def _mesh():
    return plsc.VectorSubcoreMesh(core_axis_name="c", subcore_axis_name="s")


_SC_PARAMS = pltpu.CompilerParams(use_tc_tiling_on_sc=False)


def _make_deg_kernel(n_chunks):
    @functools.partial(
        pl.kernel,
        out_type=jax.ShapeDtypeStruct((NC, NPAD), jnp.float32),
        mesh=_mesh(),
        compiler_params=_SC_PARAMS,
        scratch_types=dict(
            dst_v=pltpu.VMEM((n_chunks, CHUNK), jnp.int32),
            ones_v=pltpu.VMEM((CHUNK,), jnp.float32),
            zbuf=pltpu.VMEM((ROWS_PER_TILE,), jnp.float32),
            acc=pltpu.VMEM_SHARED((NPAD,), jnp.float32),
            sem=pltpu.SemaphoreType.DMA,
        ),
    )
    def deg_kernel(dst_hbm, deg_out, dst_v, ones_v, zbuf, acc, sem):
        cid = lax.axis_index("c")
        sid = lax.axis_index("s")
        wid = sid * NC + cid

        def fill_ones(i, _):
            ones_v[pl.ds(i * 16, 16)] = jnp.ones((16,), jnp.float32)
            zbuf[pl.ds(i * 16, 16)] = jnp.zeros((16,), jnp.float32)
            return 0

        lax.fori_loop(0, CHUNK // 16, fill_ones, 0)

        def fill_z(i, _):
            zbuf[pl.ds(i * 16, 16)] = jnp.zeros((16,), jnp.float32)
            return 0

        lax.fori_loop(CHUNK // 16, ROWS_PER_TILE // 16, fill_z, 0)

        copy_idx = pltpu.async_copy(dst_hbm.at[wid], dst_v, sem)
        pltpu.sync_copy(zbuf, acc.at[pl.ds(sid * ROWS_PER_TILE, ROWS_PER_TILE)])
        copy_idx.wait()
        plsc.subcore_barrier()

        def body(j, _):
            pltpu.sync_copy(ones_v, acc.at[dst_v.at[j]], add=True)
            return 0

        lax.fori_loop(0, n_chunks, body, 0)
        plsc.subcore_barrier()

        pltpu.sync_copy(
            acc.at[pl.ds(sid * ROWS_PER_TILE, ROWS_PER_TILE)],
            deg_out.at[cid, pl.ds(sid * ROWS_PER_TILE, ROWS_PER_TILE)],
        )

    return deg_kernel


def _make_agg_kernel(n_chunks):
    @functools.partial(
        pl.kernel,
        out_type=jax.ShapeDtypeStruct((NC, NPAD, H), jnp.float32),
        mesh=_mesh(),
        compiler_params=_SC_PARAMS,
        scratch_types=dict(
            src_v=pltpu.VMEM((n_chunks, CHUNK), jnp.int32),
            dst_v=pltpu.VMEM((n_chunks, CHUNK), jnp.int32),
            rows_v=pltpu.VMEM((3, CHUNK, H), jnp.float32),
            hs_s=pltpu.VMEM_SHARED((NPAD, H), jnp.float32),
            acc=pltpu.VMEM_SHARED((NPAD, H), jnp.float32),
            sem_g=pltpu.SemaphoreType.DMA,
            sem_s=pltpu.SemaphoreType.DMA,
            sem_i=pltpu.SemaphoreType.DMA,
        ),
    )
    def agg_kernel(hs_hbm, src_hbm, dst_hbm, p_out,
                   src_v, dst_v, rows_v, hs_s, acc, sem_g, sem_s, sem_i):
        cid = lax.axis_index("c")
        sid = lax.axis_index("s")
        wid = sid * NC + cid
        sl = pl.ds(sid * ROWS_PER_TILE, ROWS_PER_TILE)

        c_src = pltpu.async_copy(src_hbm.at[wid], src_v, sem_i)
        c_dst = pltpu.async_copy(dst_hbm.at[wid], dst_v, sem_i)
        c_hs = pltpu.async_copy(hs_hbm.at[sl], hs_s.at[sl], sem_i)

        def fill_z(r, _):
            for c4 in range(H // 16):
                rows_v[0, r, pl.ds(c4 * 16, 16)] = jnp.zeros((16,),
                                                             jnp.float32)
            return 0

        lax.fori_loop(0, CHUNK, fill_z, 0)
        for k in range(ROWS_PER_TILE // CHUNK):
            pltpu.sync_copy(
                rows_v.at[0],
                acc.at[pl.ds(sid * ROWS_PER_TILE + k * CHUNK, CHUNK)])
        c_src.wait()
        c_dst.wait()
        c_hs.wait()
        plsc.subcore_barrier()

        def gather(j):
            pltpu.async_copy(hs_s.at[src_v.at[j]],
                             rows_v.at[lax.rem(j, 3)], sem_g)

        def scatter(j):
            pltpu.async_copy(rows_v.at[lax.rem(j, 3)],
                             acc.at[dst_v.at[j]], sem_s, add=True)

        def drain(sem):
            pltpu.make_async_copy(hs_hbm.at[src_v.at[0]], rows_v.at[0],
                                  sem).wait()

        for j0 in range(2):
            gather(j0)

        def body(j, _):
            drain(sem_g)
            scatter(j)

            @pl.when(j >= 1)
            def _():
                drain(sem_s)

            @pl.when(j + 2 < n_chunks)
            def _():
                gather(j + 2)
            return 0

        lax.fori_loop(0, n_chunks, body, 0)
        drain(sem_s)
        plsc.subcore_barrier()

        pltpu.sync_copy(
            acc.at[pl.ds(sid * ROWS_PER_TILE, ROWS_PER_TILE)],
            p_out.at[cid, pl.ds(sid * ROWS_PER_TILE, ROWS_PER_TILE)],
        )

    return agg_kernel


def _tc1_body(deg_ref, x_ref, w_ref, dinv_ref, hs_ref):
    d = deg_ref[0] + deg_ref[1] + 1.0
    dinv = lax.rsqrt(d)
    dinv_ref[...] = dinv
    h = jnp.dot(x_ref[...], w_ref[...], preferred_element_type=jnp.float32)
    hs_ref[...] = h * dinv


def _tc2_body(p_ref, hs_ref, dinv_ref, b1_ref, w2_ref, out_ref):
    dinv = dinv_ref[...]
    agg = (p_ref[0] + p_ref[1] + hs_ref[...]) * dinv + b1_ref[...]
    h1 = jnp.maximum(agg, 0.0)
    out_ref[...] = jnp.dot(h1, w2_ref[...],
                           preferred_element_type=jnp.float32) * dinv


def _tc3_body(p_ref, hs_ref, dinv_ref, b2_ref, wc1_ref, bc1_ref, wc2_ref,
              bc2_ref, g_ref, bt_ref, y_ref, loss_ref, acc_ref):
    i = pl.program_id(0)
    h2 = (p_ref[0] + p_ref[1] + hs_ref[...]) * dinv_ref[...] + b2_ref[...]
    t = jnp.maximum(jnp.dot(h2, wc1_ref[...],
                            preferred_element_type=jnp.float32)
                    + bc1_ref[...], 0.0)
    o = jnp.dot(t, wc2_ref[...],
                preferred_element_type=jnp.float32) + bc2_ref[...]
    mu = jnp.mean(o, axis=1, keepdims=True)
    ctr = o - mu
    var = jnp.mean(ctr * ctr, axis=1, keepdims=True)
    o = ctr * lax.rsqrt(var + 1e-5) * g_ref[...] + bt_ref[...]
    m = jnp.max(o, axis=1, keepdims=True)
    lse = m + jnp.log(jnp.sum(jnp.exp(o - m), axis=1, keepdims=True))
    cls = lax.broadcasted_iota(jnp.int32, (BN, C), 1)
    tgt = jnp.sum(jnp.where(cls == y_ref[...], o, 0.0), axis=1, keepdims=True)
    rows = i * BN + lax.broadcasted_iota(jnp.int32, (BN, 1), 0)
    part = jnp.sum(jnp.where(rows < N, tgt - lse, 0.0))
    tot = jnp.where(i == 0, 0.0, acc_ref[0, 0]) + part
    acc_ref[0, 0] = tot

    @pl.when(i == GRID - 1)
    def _():
        loss_ref[0, 0] = -tot / N


def kernel(x, edge_index, y, W1, b1, W2, b2, Wc1, bc1, Wc2, bc2, gamma, beta):
    e = edge_index.shape[1]
    e_per_tile = -(-e // (NW * CHUNK)) * CHUNK
    n_chunks = e_per_tile // CHUNK
    epad = e_per_tile * NW

    x_pad = jnp.zeros((NPAD, D), jnp.float32).at[:N].set(x)
    src = jnp.full((epad,), NPAD - 1, jnp.int32).at[:e].set(
        edge_index[0].astype(jnp.int32))
    dst = jnp.full((epad,), NPAD - 1, jnp.int32).at[:e].set(
        edge_index[1].astype(jnp.int32))
    src_r = src.reshape(NW, n_chunks, CHUNK)
    dst_r = dst.reshape(NW, n_chunks, CHUNK)
    y_pad = jnp.zeros((NPAD, 1), jnp.int32).at[:N].set(y.astype(jnp.int32))

    deg_p = _make_deg_kernel(n_chunks)(dst_r)
    deg_p = deg_p.reshape(NC, NPAD, 1)

    dinv, hs1 = pl.pallas_call(
        _tc1_body,
        grid=(GRID,),
        in_specs=[
            pl.BlockSpec((NC, BN, 1), lambda i: (0, i, 0)),
            pl.BlockSpec((BN, D), lambda i: (i, 0)),
            pl.BlockSpec((D, H), lambda i: (0, 0)),
        ],
        out_specs=[
            pl.BlockSpec((BN, 1), lambda i: (i, 0)),
            pl.BlockSpec((BN, H), lambda i: (i, 0)),
        ],
        out_shape=[
            jax.ShapeDtypeStruct((NPAD, 1), jnp.float32),
            jax.ShapeDtypeStruct((NPAD, H), jnp.float32),
        ],
    )(deg_p, x_pad, W1)

    agg = _make_agg_kernel(n_chunks)

    p1 = agg(hs1, src_r, dst_r)

    hs2 = pl.pallas_call(
        _tc2_body,
        grid=(GRID,),
        in_specs=[
            pl.BlockSpec((NC, BN, H), lambda i: (0, i, 0)),
            pl.BlockSpec((BN, H), lambda i: (i, 0)),
            pl.BlockSpec((BN, 1), lambda i: (i, 0)),
            pl.BlockSpec((1, H), lambda i: (0, 0)),
            pl.BlockSpec((H, H), lambda i: (0, 0)),
        ],
        out_specs=pl.BlockSpec((BN, H), lambda i: (i, 0)),
        out_shape=jax.ShapeDtypeStruct((NPAD, H), jnp.float32),
    )(p1, hs1, dinv, b1.reshape(1, H), W2)

    p2 = agg(hs2, src_r, dst_r)

    loss = pl.pallas_call(
        _tc3_body,
        grid=(GRID,),
        in_specs=[
            pl.BlockSpec((NC, BN, H), lambda i: (0, i, 0)),
            pl.BlockSpec((BN, H), lambda i: (i, 0)),
            pl.BlockSpec((BN, 1), lambda i: (i, 0)),
            pl.BlockSpec((1, H), lambda i: (0, 0)),
            pl.BlockSpec((H, H), lambda i: (0, 0)),
            pl.BlockSpec((1, H), lambda i: (0, 0)),
            pl.BlockSpec((H, C), lambda i: (0, 0)),
            pl.BlockSpec((1, C), lambda i: (0, 0)),
            pl.BlockSpec((1, C), lambda i: (0, 0)),
            pl.BlockSpec((1, C), lambda i: (0, 0)),
            pl.BlockSpec((BN, 1), lambda i: (i, 0)),
        ],
        out_specs=pl.BlockSpec((1, 1), lambda i: (0, 0),
                               memory_space=pltpu.SMEM),
        out_shape=jax.ShapeDtypeStruct((1, 1), jnp.float32),
        scratch_shapes=[pltpu.SMEM((1, 1), jnp.float32)],
    )(p2, hs2, dinv, b2.reshape(1, H), Wc1, bc1.reshape(1, H), Wc2,
      bc2.reshape(1, C), gamma.reshape(1, C), beta.reshape(1, C), y_pad)

    return loss[0, 0]


# speedup vs baseline: 50.1966x; 1.3831x over previous
import functools

import jax
import jax.numpy as jnp
from jax import lax
from jax.experimental import pallas as pl
from jax.experimental.pallas import tpu as pltpu
from jax.experimental.pallas import tpu_sc as plsc

N = 10000
D = 128
H = 64
C = 40

NC = 2
NS = 16
NW = NC * NS

NPAD = 10240
ROWS_PER_TILE = NPAD // NS
CHUNK = 128

BN = 1024
GRID = NPAD // BN


def _mesh():
    return plsc.VectorSubcoreMesh(core_axis_name="c", subcore_axis_name="s")


_SC_PARAMS = pltpu.CompilerParams(use_tc_tiling_on_sc=False)


def _make_deg_kernel(n_chunks):
    @functools.partial(
        pl.kernel,
        out_type=jax.ShapeDtypeStruct((NC, NPAD), jnp.float32),
        mesh=_mesh(),
        compiler_params=_SC_PARAMS,
        scratch_types=dict(
            dst_v=pltpu.VMEM((n_chunks, CHUNK), jnp.int32),
            ones_v=pltpu.VMEM((CHUNK,), jnp.float32),
            zbuf=pltpu.VMEM((ROWS_PER_TILE,), jnp.float32),
            acc=pltpu.VMEM_SHARED((NPAD,), jnp.float32),
            sem=pltpu.SemaphoreType.DMA,
        ),
    )
    def deg_kernel(dst_hbm, deg_out, dst_v, ones_v, zbuf, acc, sem):
        cid = lax.axis_index("c")
        sid = lax.axis_index("s")
        wid = sid * NC + cid

        def fill_ones(i, _):
            ones_v[pl.ds(i * 16, 16)] = jnp.ones((16,), jnp.float32)
            zbuf[pl.ds(i * 16, 16)] = jnp.zeros((16,), jnp.float32)
            return 0

        lax.fori_loop(0, CHUNK // 16, fill_ones, 0)

        def fill_z(i, _):
            zbuf[pl.ds(i * 16, 16)] = jnp.zeros((16,), jnp.float32)
            return 0

        lax.fori_loop(CHUNK // 16, ROWS_PER_TILE // 16, fill_z, 0)

        copy_idx = pltpu.async_copy(dst_hbm.at[wid], dst_v, sem)
        pltpu.sync_copy(zbuf, acc.at[pl.ds(sid * ROWS_PER_TILE, ROWS_PER_TILE)])
        copy_idx.wait()
        plsc.subcore_barrier()

        def body(j, _):
            pltpu.sync_copy(ones_v, acc.at[dst_v.at[j]], add=True)
            return 0

        lax.fori_loop(0, n_chunks, body, 0)
        plsc.subcore_barrier()

        pltpu.sync_copy(
            acc.at[pl.ds(sid * ROWS_PER_TILE, ROWS_PER_TILE)],
            deg_out.at[cid, pl.ds(sid * ROWS_PER_TILE, ROWS_PER_TILE)],
        )

    return deg_kernel


def _make_agg_kernel(n_chunks):
    @functools.partial(
        pl.kernel,
        out_type=jax.ShapeDtypeStruct((NC, NPAD, H), jnp.float32),
        mesh=_mesh(),
        compiler_params=_SC_PARAMS,
        scratch_types=dict(
            src_v=pltpu.VMEM((n_chunks, CHUNK), jnp.int32),
            dst_v=pltpu.VMEM((n_chunks, CHUNK), jnp.int32),
            rows_v=pltpu.VMEM((3, CHUNK, H), jnp.float32),
            hs_s=pltpu.VMEM_SHARED((NPAD, H), jnp.float32),
            acc=pltpu.VMEM_SHARED((NPAD, H), jnp.float32),
            sem_g=pltpu.SemaphoreType.DMA,
            sem_s=pltpu.SemaphoreType.DMA,
            sem_i=pltpu.SemaphoreType.DMA,
        ),
    )
    def agg_kernel(hs_hbm, src_hbm, dst_hbm, p_out,
                   src_v, dst_v, rows_v, hs_s, acc, sem_g, sem_s, sem_i):
        cid = lax.axis_index("c")
        sid = lax.axis_index("s")
        wid = sid * NC + cid
        sl = pl.ds(sid * ROWS_PER_TILE, ROWS_PER_TILE)

        c_src = pltpu.async_copy(src_hbm.at[wid], src_v, sem_i)
        c_dst = pltpu.async_copy(dst_hbm.at[wid], dst_v, sem_i)
        c_hs = pltpu.async_copy(hs_hbm.at[sl], hs_s.at[sl], sem_i)

        def fill_z(r, _):
            for c4 in range(H // 16):
                rows_v[0, r, pl.ds(c4 * 16, 16)] = jnp.zeros((16,),
                                                             jnp.float32)
            return 0

        lax.fori_loop(0, CHUNK, fill_z, 0)
        for k in range(ROWS_PER_TILE // CHUNK):
            pltpu.sync_copy(
                rows_v.at[0],
                acc.at[pl.ds(sid * ROWS_PER_TILE + k * CHUNK, CHUNK)])
        c_src.wait()
        c_dst.wait()
        c_hs.wait()
        plsc.subcore_barrier()

        def gather(j):
            pltpu.async_copy(hs_s.at[src_v.at[j]],
                             rows_v.at[lax.rem(j, 3)], sem_g)

        def scatter(j):
            pltpu.async_copy(rows_v.at[lax.rem(j, 3)],
                             acc.at[dst_v.at[j]], sem_s, add=True)

        def drain(sem):
            pltpu.make_async_copy(hs_hbm.at[src_v.at[0]], rows_v.at[0],
                                  sem).wait()

        for j0 in range(2):
            gather(j0)

        def body(j, _):
            drain(sem_g)
            scatter(j)

            @pl.when(j >= 1)
            def _():
                drain(sem_s)

            @pl.when(j + 2 < n_chunks)
            def _():
                gather(j + 2)
            return 0

        lax.fori_loop(0, n_chunks, body, 0)
        drain(sem_s)
        plsc.subcore_barrier()

        pltpu.sync_copy(
            acc.at[pl.ds(sid * ROWS_PER_TILE, ROWS_PER_TILE)],
            p_out.at[cid, pl.ds(sid * ROWS_PER_TILE, ROWS_PER_TILE)],
        )

    return agg_kernel


BN2 = BN // 2


def _halves(v):
    return v[:, :H], v[:, H:]


def _node_scalar(dlane, par):
    n = 2 * lax.broadcasted_iota(jnp.int32, (BN2, 1), 0) + par
    rowm = (lax.broadcasted_iota(jnp.int32, (BN2, BN // 128), 1)
            == (n >> 7)).astype(jnp.float32)
    t = jnp.dot(rowm, dlane, preferred_element_type=jnp.float32)
    lanem = lax.broadcasted_iota(jnp.int32, (BN2, 128), 1) == (n & 127)
    return jnp.sum(jnp.where(lanem, t, 0.0), axis=1, keepdims=True)


def _tc1_body(deg_ref, x_ref, w_ref, dinv_ref, hs_ref):
    d = deg_ref[0] + deg_ref[1] + 1.0
    da = jnp.broadcast_to(lax.rsqrt(_node_scalar(d, 0)), (BN2, H))
    db = jnp.broadcast_to(lax.rsqrt(_node_scalar(d, 1)), (BN2, H))
    dinv_ref[...] = jnp.concatenate([da, db], axis=1)
    xa, xb = x_ref[...][:, :D], x_ref[...][:, D:]
    w = w_ref[...]
    ha = jnp.dot(xa, w, preferred_element_type=jnp.float32) * da
    hb = jnp.dot(xb, w, preferred_element_type=jnp.float32) * db
    hs_ref[...] = jnp.concatenate([ha, hb], axis=1)


def _tc2_body(p_ref, hs_ref, dinv_ref, b1_ref, w2_ref, out_ref):
    dinv = dinv_ref[...]
    agg = (p_ref[0] + p_ref[1] + hs_ref[...]) * dinv + b1_ref[...]
    h1 = jnp.maximum(agg, 0.0)
    h1a, h1b = _halves(h1)
    w = w2_ref[...]
    oa = jnp.dot(h1a, w, preferred_element_type=jnp.float32)
    ob = jnp.dot(h1b, w, preferred_element_type=jnp.float32)
    out_ref[...] = jnp.concatenate([oa, ob], axis=1) * dinv


def _head_half(o, g, bt, y, nbase):
    mu = jnp.mean(o, axis=1, keepdims=True)
    ctr = o - mu
    var = jnp.mean(ctr * ctr, axis=1, keepdims=True)
    o = ctr * lax.rsqrt(var + 1e-5) * g + bt
    m = jnp.max(o, axis=1, keepdims=True)
    lse = m + jnp.log(jnp.sum(jnp.exp(o - m), axis=1, keepdims=True))
    cls = lax.broadcasted_iota(jnp.int32, (BN2, C), 1).astype(jnp.float32)
    tgt = jnp.sum(jnp.where(cls == y, o, 0.0), axis=1, keepdims=True)
    rows = nbase + 2 * lax.broadcasted_iota(jnp.int32, (BN2, 1), 0)
    return jnp.sum(jnp.where(rows < N, tgt - lse, 0.0))


def _tc3_body(p_ref, hs_ref, dinv_ref, b2_ref, wc1_ref, bc1_ref, wc2_ref,
              bc2_ref, g_ref, bt_ref, y_ref, loss_ref, acc_ref):
    i = pl.program_id(0)
    h2 = (p_ref[0] + p_ref[1] + hs_ref[...]) * dinv_ref[...] + b2_ref[...]
    h2a, h2b = _halves(h2)
    wc1 = wc1_ref[...]
    wc2 = wc2_ref[...]
    g = g_ref[...]
    bt = bt_ref[...]
    yf = y_ref[...].astype(jnp.float32)
    part = 0.0
    for k, hh in enumerate((h2a, h2b)):
        t = jnp.maximum(jnp.dot(hh, wc1, preferred_element_type=jnp.float32)
                        + bc1_ref[...], 0.0)
        o = jnp.dot(t, wc2, preferred_element_type=jnp.float32) + bc2_ref[...]
        part += _head_half(o, g, bt, _node_scalar(yf, k), i * BN + k)
    tot = jnp.where(i == 0, 0.0, acc_ref[0, 0]) + part
    acc_ref[0, 0] = tot

    @pl.when(i == GRID - 1)
    def _():
        loss_ref[0, 0] = -tot / N


def kernel(x, edge_index, y, W1, b1, W2, b2, Wc1, bc1, Wc2, bc2, gamma, beta):
    e = edge_index.shape[1]
    e_per_tile = -(-e // (NW * CHUNK)) * CHUNK
    n_chunks = e_per_tile // CHUNK
    epad = e_per_tile * NW

    NP2 = NPAD // 2
    xv = jnp.pad(x.reshape(N // 2, 2 * D).astype(jnp.float32),
                 ((0, NP2 - N // 2), (0, 0)))
    er = jnp.pad(edge_index.astype(jnp.int32), ((0, 0), (0, epad - e)),
                 constant_values=NPAD - 1)
    src_r = er[0].reshape(NW, n_chunks, CHUNK)
    dst_r = er[1].reshape(NW, n_chunks, CHUNK)
    yv = jnp.pad(y.astype(jnp.int32).reshape(N), (0, NPAD - N)
                 ).reshape(NPAD // 128, 128)

    deg_p = _make_deg_kernel(n_chunks)(dst_r)
    deg_v = deg_p.reshape(NC, NPAD // 128, 128)

    dinvv, hs1v = pl.pallas_call(
        _tc1_body,
        grid=(GRID,),
        in_specs=[
            pl.BlockSpec((NC, BN // 128, 128), lambda i: (0, i, 0)),
            pl.BlockSpec((BN2, 2 * D), lambda i: (i, 0)),
            pl.BlockSpec((D, H), lambda i: (0, 0)),
        ],
        out_specs=[
            pl.BlockSpec((BN2, 2 * H), lambda i: (i, 0)),
            pl.BlockSpec((BN2, 2 * H), lambda i: (i, 0)),
        ],
        out_shape=[
            jax.ShapeDtypeStruct((NP2, 2 * H), jnp.float32),
            jax.ShapeDtypeStruct((NP2, 2 * H), jnp.float32),
        ],
    )(deg_v, xv, W1)

    agg = _make_agg_kernel(n_chunks)

    def paired_specs():
        return [
            pl.BlockSpec((NC, BN2, 2 * H), lambda i: (0, i, 0)),
            pl.BlockSpec((BN2, 2 * H), lambda i: (i, 0)),
            pl.BlockSpec((BN2, 2 * H), lambda i: (i, 0)),
        ]

    p1 = agg(hs1v.reshape(NPAD, H), src_r, dst_r)

    hs2v = pl.pallas_call(
        _tc2_body,
        grid=(GRID,),
        in_specs=paired_specs() + [
            pl.BlockSpec((1, 2 * H), lambda i: (0, 0)),
            pl.BlockSpec((H, H), lambda i: (0, 0)),
        ],
        out_specs=pl.BlockSpec((BN2, 2 * H), lambda i: (i, 0)),
        out_shape=jax.ShapeDtypeStruct((NP2, 2 * H), jnp.float32),
    )(p1.reshape(NC, NP2, 2 * H), hs1v, dinvv,
      jnp.tile(b1.reshape(1, H), (1, 2)), W2)

    p2 = agg(hs2v.reshape(NPAD, H), src_r, dst_r)

    loss = pl.pallas_call(
        _tc3_body,
        grid=(GRID,),
        in_specs=paired_specs() + [
            pl.BlockSpec((1, 2 * H), lambda i: (0, 0)),
            pl.BlockSpec((H, H), lambda i: (0, 0)),
            pl.BlockSpec((1, H), lambda i: (0, 0)),
            pl.BlockSpec((H, C), lambda i: (0, 0)),
            pl.BlockSpec((1, C), lambda i: (0, 0)),
            pl.BlockSpec((1, C), lambda i: (0, 0)),
            pl.BlockSpec((1, C), lambda i: (0, 0)),
            pl.BlockSpec((BN // 128, 128), lambda i: (i, 0)),
        ],
        out_specs=pl.BlockSpec((1, 1), lambda i: (0, 0),
                               memory_space=pltpu.SMEM),
        out_shape=jax.ShapeDtypeStruct((1, 1), jnp.float32),
        scratch_shapes=[pltpu.SMEM((1, 1), jnp.float32)],
    )(p2.reshape(NC, NP2, 2 * H), hs2v, dinvv,
      jnp.tile(b2.reshape(1, H), (1, 2)), Wc1, bc1.reshape(1, H), Wc2,
      bc2.reshape(1, C), gamma.reshape(1, C), beta.reshape(1, C), yv)

    return loss[0, 0]


# speedup vs baseline: 58.2218x; 1.1599x over previous
import functools

import jax
import jax.numpy as jnp
from jax import lax
from jax.experimental import pallas as pl
from jax.experimental.pallas import tpu as pltpu
from jax.experimental.pallas import tpu_sc as plsc

N = 10000
D = 128
H = 64
C = 40

NC = 2
NS = 16
NW = NC * NS

NPAD = 10240
ROWS_PER_TILE = NPAD // NS
CHUNK = 128

BN = 1024
GRID = NPAD // BN


def _mesh():
    return plsc.VectorSubcoreMesh(core_axis_name="c", subcore_axis_name="s")


_SC_PARAMS = pltpu.CompilerParams(use_tc_tiling_on_sc=False)


def _make_deg_kernel(n_chunks):
    @functools.partial(
        pl.kernel,
        out_type=jax.ShapeDtypeStruct((NC, NPAD), jnp.float32),
        mesh=_mesh(),
        compiler_params=_SC_PARAMS,
        scratch_types=dict(
            dst_v=pltpu.VMEM((n_chunks, CHUNK), jnp.int32),
            ones_v=pltpu.VMEM((CHUNK,), jnp.float32),
            zbuf=pltpu.VMEM((ROWS_PER_TILE,), jnp.float32),
            acc=pltpu.VMEM_SHARED((NPAD,), jnp.float32),
            sem=pltpu.SemaphoreType.DMA,
        ),
    )
    def deg_kernel(dst_hbm, deg_out, dst_v, ones_v, zbuf, acc, sem):
        cid = lax.axis_index("c")
        sid = lax.axis_index("s")
        wid = sid * NC + cid

        def fill_ones(i, _):
            ones_v[pl.ds(i * 16, 16)] = jnp.ones((16,), jnp.float32)
            zbuf[pl.ds(i * 16, 16)] = jnp.zeros((16,), jnp.float32)
            return 0

        lax.fori_loop(0, CHUNK // 16, fill_ones, 0)

        def fill_z(i, _):
            zbuf[pl.ds(i * 16, 16)] = jnp.zeros((16,), jnp.float32)
            return 0

        lax.fori_loop(CHUNK // 16, ROWS_PER_TILE // 16, fill_z, 0)

        copy_idx = pltpu.async_copy(dst_hbm.at[wid], dst_v, sem)
        pltpu.sync_copy(zbuf, acc.at[pl.ds(sid * ROWS_PER_TILE, ROWS_PER_TILE)])
        copy_idx.wait()
        plsc.subcore_barrier()

        def body(j, _):
            pltpu.sync_copy(ones_v, acc.at[dst_v.at[j]], add=True)
            return 0

        lax.fori_loop(0, n_chunks, body, 0)
        plsc.subcore_barrier()

        pltpu.sync_copy(
            acc.at[pl.ds(sid * ROWS_PER_TILE, ROWS_PER_TILE)],
            deg_out.at[cid, pl.ds(sid * ROWS_PER_TILE, ROWS_PER_TILE)],
        )

    return deg_kernel


def _make_agg_kernel(n_chunks):
    @functools.partial(
        pl.kernel,
        out_type=jax.ShapeDtypeStruct((NC, NPAD, H), jnp.bfloat16),
        mesh=_mesh(),
        compiler_params=_SC_PARAMS,
        scratch_types=dict(
            src_v=pltpu.VMEM((n_chunks, CHUNK), jnp.int32),
            dst_v=pltpu.VMEM((n_chunks, CHUNK), jnp.int32),
            rows_v=pltpu.VMEM((3, CHUNK, H), jnp.bfloat16),
            hs_s=pltpu.VMEM_SHARED((NPAD, H), jnp.bfloat16),
            acc=pltpu.VMEM_SHARED((NPAD, H), jnp.bfloat16),
            sem_g=pltpu.SemaphoreType.DMA,
            sem_s=pltpu.SemaphoreType.DMA,
            sem_i=pltpu.SemaphoreType.DMA,
        ),
    )
    def agg_kernel(hs_hbm, src_hbm, dst_hbm, p_out,
                   src_v, dst_v, rows_v, hs_s, acc, sem_g, sem_s, sem_i):
        cid = lax.axis_index("c")
        sid = lax.axis_index("s")
        wid = sid * NC + cid
        sl = pl.ds(sid * ROWS_PER_TILE, ROWS_PER_TILE)

        c_src = pltpu.async_copy(src_hbm.at[wid], src_v, sem_i)
        c_dst = pltpu.async_copy(dst_hbm.at[wid], dst_v, sem_i)
        c_hs = pltpu.async_copy(hs_hbm.at[sl], hs_s.at[sl], sem_i)

        def fill_z(r, _):
            for c4 in range(H // 32):
                rows_v[0, r, pl.ds(c4 * 32, 32)] = jnp.zeros((32,),
                                                             jnp.bfloat16)
            return 0

        lax.fori_loop(0, CHUNK, fill_z, 0)
        for k in range(ROWS_PER_TILE // CHUNK):
            pltpu.sync_copy(
                rows_v.at[0],
                acc.at[pl.ds(sid * ROWS_PER_TILE + k * CHUNK, CHUNK)])
        c_src.wait()
        c_dst.wait()
        c_hs.wait()
        plsc.subcore_barrier()

        def gather(j):
            pltpu.async_copy(hs_s.at[src_v.at[j]],
                             rows_v.at[lax.rem(j, 3)], sem_g)

        def scatter(j):
            pltpu.async_copy(rows_v.at[lax.rem(j, 3)],
                             acc.at[dst_v.at[j]], sem_s, add=True)

        def drain(sem):
            pltpu.make_async_copy(hs_hbm.at[src_v.at[0]], rows_v.at[0],
                                  sem).wait()

        for j0 in range(2):
            gather(j0)

        def body(j, _):
            drain(sem_g)
            scatter(j)

            @pl.when(j >= 1)
            def _():
                drain(sem_s)

            @pl.when(j + 2 < n_chunks)
            def _():
                gather(j + 2)
            return 0

        lax.fori_loop(0, n_chunks, body, 0)
        drain(sem_s)
        plsc.subcore_barrier()

        pltpu.sync_copy(
            acc.at[pl.ds(sid * ROWS_PER_TILE, ROWS_PER_TILE)],
            p_out.at[cid, pl.ds(sid * ROWS_PER_TILE, ROWS_PER_TILE)],
        )

    return agg_kernel


BN2 = BN // 2


def _halves(v):
    return v[:, :H], v[:, H:]


def _node_scalar(dlane, par):
    n = 2 * lax.broadcasted_iota(jnp.int32, (BN2, 1), 0) + par
    rowm = (lax.broadcasted_iota(jnp.int32, (BN2, BN // 128), 1)
            == (n >> 7)).astype(jnp.float32)
    t = jnp.dot(rowm, dlane, preferred_element_type=jnp.float32)
    lanem = lax.broadcasted_iota(jnp.int32, (BN2, 128), 1) == (n & 127)
    return jnp.sum(jnp.where(lanem, t, 0.0), axis=1, keepdims=True)


def _tc1_body(deg_ref, x_ref, w_ref, dinv_ref, hs_ref):
    d = deg_ref[0] + deg_ref[1] + 1.0
    da = jnp.broadcast_to(lax.rsqrt(_node_scalar(d, 0)), (BN2, H))
    db = jnp.broadcast_to(lax.rsqrt(_node_scalar(d, 1)), (BN2, H))
    dinv_ref[...] = jnp.concatenate([da, db], axis=1)
    xa, xb = x_ref[...][:, :D], x_ref[...][:, D:]
    w = w_ref[...]
    ha = jnp.dot(xa, w, preferred_element_type=jnp.float32) * da
    hb = jnp.dot(xb, w, preferred_element_type=jnp.float32) * db
    hs_ref[...] = jnp.concatenate([ha, hb], axis=1).astype(jnp.bfloat16)


def _agg_f32(p_ref, hs_ref):
    return (p_ref[0].astype(jnp.float32) + p_ref[1].astype(jnp.float32)
            + hs_ref[...].astype(jnp.float32))


def _tc2_body(p_ref, hs_ref, dinv_ref, b1_ref, w2_ref, out_ref):
    dinv = dinv_ref[...]
    agg = _agg_f32(p_ref, hs_ref) * dinv + b1_ref[...]
    h1 = jnp.maximum(agg, 0.0)
    h1a, h1b = _halves(h1)
    w = w2_ref[...]
    oa = jnp.dot(h1a, w, preferred_element_type=jnp.float32)
    ob = jnp.dot(h1b, w, preferred_element_type=jnp.float32)
    out_ref[...] = (jnp.concatenate([oa, ob], axis=1)
                    * dinv).astype(jnp.bfloat16)


def _head_half(o, g, bt, y, nbase):
    mu = jnp.mean(o, axis=1, keepdims=True)
    ctr = o - mu
    var = jnp.mean(ctr * ctr, axis=1, keepdims=True)
    o = ctr * lax.rsqrt(var + 1e-5) * g + bt
    m = jnp.max(o, axis=1, keepdims=True)
    lse = m + jnp.log(jnp.sum(jnp.exp(o - m), axis=1, keepdims=True))
    cls = lax.broadcasted_iota(jnp.int32, (BN2, C), 1).astype(jnp.float32)
    tgt = jnp.sum(jnp.where(cls == y, o, 0.0), axis=1, keepdims=True)
    rows = nbase + 2 * lax.broadcasted_iota(jnp.int32, (BN2, 1), 0)
    return jnp.sum(jnp.where(rows < N, tgt - lse, 0.0))


def _tc3_body(p_ref, hs_ref, dinv_ref, b2_ref, wc1_ref, bc1_ref, wc2_ref,
              bc2_ref, g_ref, bt_ref, y_ref, loss_ref, acc_ref):
    i = pl.program_id(0)
    h2 = _agg_f32(p_ref, hs_ref) * dinv_ref[...] + b2_ref[...]
    h2a, h2b = _halves(h2)
    wc1 = wc1_ref[...]
    wc2 = wc2_ref[...]
    g = g_ref[...]
    bt = bt_ref[...]
    yf = y_ref[...].astype(jnp.float32)
    part = 0.0
    for k, hh in enumerate((h2a, h2b)):
        t = jnp.maximum(jnp.dot(hh, wc1, preferred_element_type=jnp.float32)
                        + bc1_ref[...], 0.0)
        o = jnp.dot(t, wc2, preferred_element_type=jnp.float32) + bc2_ref[...]
        part += _head_half(o, g, bt, _node_scalar(yf, k), i * BN + k)
    tot = jnp.where(i == 0, 0.0, acc_ref[0, 0]) + part
    acc_ref[0, 0] = tot

    @pl.when(i == GRID - 1)
    def _():
        loss_ref[0, 0] = -tot / N


def kernel(x, edge_index, y, W1, b1, W2, b2, Wc1, bc1, Wc2, bc2, gamma, beta):
    e = edge_index.shape[1]
    e_per_tile = -(-e // (NW * CHUNK)) * CHUNK
    n_chunks = e_per_tile // CHUNK
    epad = e_per_tile * NW

    NP2 = NPAD // 2
    xv = jnp.pad(x.reshape(N // 2, 2 * D).astype(jnp.float32),
                 ((0, NP2 - N // 2), (0, 0)))
    er = jnp.pad(edge_index.astype(jnp.int32), ((0, 0), (0, epad - e)),
                 constant_values=NPAD - 1)
    src_r = er[0].reshape(NW, n_chunks, CHUNK)
    dst_r = er[1].reshape(NW, n_chunks, CHUNK)
    yv = jnp.pad(y.astype(jnp.int32).reshape(N), (0, NPAD - N)
                 ).reshape(NPAD // 128, 128)

    deg_p = _make_deg_kernel(n_chunks)(dst_r)
    deg_v = deg_p.reshape(NC, NPAD // 128, 128)

    dinvv, hs1v = pl.pallas_call(
        _tc1_body,
        grid=(GRID,),
        in_specs=[
            pl.BlockSpec((NC, BN // 128, 128), lambda i: (0, i, 0)),
            pl.BlockSpec((BN2, 2 * D), lambda i: (i, 0)),
            pl.BlockSpec((D, H), lambda i: (0, 0)),
        ],
        out_specs=[
            pl.BlockSpec((BN2, 2 * H), lambda i: (i, 0)),
            pl.BlockSpec((BN2, 2 * H), lambda i: (i, 0)),
        ],
        out_shape=[
            jax.ShapeDtypeStruct((NP2, 2 * H), jnp.float32),
            jax.ShapeDtypeStruct((NP2, 2 * H), jnp.bfloat16),
        ],
    )(deg_v, xv, W1)

    agg = _make_agg_kernel(n_chunks)

    def paired_specs():
        return [
            pl.BlockSpec((NC, BN2, 2 * H), lambda i: (0, i, 0)),
            pl.BlockSpec((BN2, 2 * H), lambda i: (i, 0)),
            pl.BlockSpec((BN2, 2 * H), lambda i: (i, 0)),
        ]

    p1 = agg(hs1v.reshape(NPAD, H), src_r, dst_r)

    hs2v = pl.pallas_call(
        _tc2_body,
        grid=(GRID,),
        in_specs=paired_specs() + [
            pl.BlockSpec((1, 2 * H), lambda i: (0, 0)),
            pl.BlockSpec((H, H), lambda i: (0, 0)),
        ],
        out_specs=pl.BlockSpec((BN2, 2 * H), lambda i: (i, 0)),
        out_shape=jax.ShapeDtypeStruct((NP2, 2 * H), jnp.bfloat16),
    )(p1.reshape(NC, NP2, 2 * H), hs1v, dinvv,
      jnp.tile(b1.reshape(1, H), (1, 2)), W2)

    p2 = agg(hs2v.reshape(NPAD, H), src_r, dst_r)

    loss = pl.pallas_call(
        _tc3_body,
        grid=(GRID,),
        in_specs=paired_specs() + [
            pl.BlockSpec((1, 2 * H), lambda i: (0, 0)),
            pl.BlockSpec((H, H), lambda i: (0, 0)),
            pl.BlockSpec((1, H), lambda i: (0, 0)),
            pl.BlockSpec((H, C), lambda i: (0, 0)),
            pl.BlockSpec((1, C), lambda i: (0, 0)),
            pl.BlockSpec((1, C), lambda i: (0, 0)),
            pl.BlockSpec((1, C), lambda i: (0, 0)),
            pl.BlockSpec((BN // 128, 128), lambda i: (i, 0)),
        ],
        out_specs=pl.BlockSpec((1, 1), lambda i: (0, 0),
                               memory_space=pltpu.SMEM),
        out_shape=jax.ShapeDtypeStruct((1, 1), jnp.float32),
        scratch_shapes=[pltpu.SMEM((1, 1), jnp.float32)],
    )(p2.reshape(NC, NP2, 2 * H), hs2v, dinvv,
      jnp.tile(b2.reshape(1, H), (1, 2)), Wc1, bc1.reshape(1, H), Wc2,
      bc2.reshape(1, C), gamma.reshape(1, C), beta.reshape(1, C), yv)

    return loss[0, 0]
